# baseline (speedup 1.0000x reference)
import jax
import jax.numpy as jnp
from jax import lax
from jax.experimental import pallas as pl
from jax.experimental.pallas import tpu as pltpu
from jax.experimental.pallas import tpu_sc as plsc

N, C, H, W = 4, 96, 224, 224
HW = H * W
L = 16
NC, NS = 2, 16
CPAIRS = C // 2
TASKS_PER_TILE = (2 * CPAIRS) // NS
CHUNK_ROWS = 8
ROW_VECS = W // L
CHUNK = CHUNK_ROWS * W
CVECS = CHUNK // L
NCHUNKS = H // CHUNK_ROWS
SCJOBS = 2 * NCHUNKS
SX = float(W) / float(W - 1)
SY = float(H) / float(H - 1)


def _floor_i32(p):
    t = p.astype(jnp.int32)
    tf = t.astype(jnp.float32)
    return jnp.where(tf > p, t - 1, t)


def _sc_body(x_hbm, gxy_hbm, out_hbm, tbl_hbm, img_v, g_v, st_v,
             ta_v, tb_v, oa0_v, oa1_v, ob0_v, ob1_v,
             simg, sta, stb, soa, sob):
    cid = lax.axis_index("c")
    sid = lax.axis_index("s")

    def task_chan(t):
        tt = sid * TASKS_PER_TILE + t
        n_local = tt // CPAIRS
        cp = tt % CPAIRS
        n = cid * 2 + n_local
        return n, n * C + cp * 2

    def img_src(chan):
        return x_hbm.at[pl.ds(chan * HW, 2 * HW)]

    _, chan0 = task_chan(0)
    pltpu.async_copy(img_src(chan0), img_v, simg)

    def table_job(jj):
        n_local = jj // NCHUNKS
        ck = jj % NCHUNKS
        n = cid * 2 + n_local
        pltpu.sync_copy(
            gxy_hbm.at[pl.ds((n * NCHUNKS + ck) * 2 * CHUNK, 2 * CHUNK)],
            g_v)

        def row_fn(ri, carry2):
            i = ck * CHUNK_ROWS + ri
            i_f = lax.convert_element_type(i, jnp.float32)

            @plsc.parallel_loop(0, ROW_VECS, unroll=2)
            def vec_fn(v):
                off = ri * W + v * L
                gxv = g_v[pl.ds(off, L)]
                gyv = g_v[pl.ds(CHUNK + off, L)]
                jb = lax.convert_element_type(v * L, jnp.float32)
                jf = lax.iota(jnp.int32, L).astype(jnp.float32) + jb
                px = (gxv + jf) * SX - 0.5
                py = (gyv + i_f) * SY - 0.5
                px = jnp.minimum(jnp.maximum(px, -2.0), float(W + 2))
                py = jnp.minimum(jnp.maximum(py, -2.0), float(H + 2))
                x0 = _floor_i32(px)
                y0 = _floor_i32(py)
                wx1 = px - x0.astype(jnp.float32)
                wx0 = 1.0 - wx1
                wy1 = py - y0.astype(jnp.float32)
                wy0 = 1.0 - wy1
                zero = jnp.zeros((L,), jnp.float32)
                wx0 = jnp.where((x0 >= 0) & (x0 <= W - 1), wx0, zero)
                wx1 = jnp.where((x0 >= -1) & (x0 <= W - 2), wx1, zero)
                wy0 = jnp.where((y0 >= 0) & (y0 <= H - 1), wy0, zero)
                wy1 = jnp.where((y0 >= -1) & (y0 <= H - 2), wy1, zero)
                xc0 = jnp.maximum(jnp.minimum(x0, W - 1), 0)
                xc1 = jnp.maximum(jnp.minimum(x0 + 1, W - 1), 0)
                yb0 = jnp.maximum(jnp.minimum(y0, H - 1), 0) * W
                yb1 = jnp.maximum(jnp.minimum(y0 + 1, H - 1), 0) * W
                ipk = (yb0 + xc0 + lax.shift_left(xc1 - xc0, 16)
                       + lax.shift_left(yb1 - yb0, 17))
                wxp = plsc.pack(wx0, wx1, format=plsc.PackFormat.INTERLEAVED)
                wyp = plsc.pack(wy0, wy1, format=plsc.PackFormat.INTERLEAVED)
                st_v[pl.ds(0 * CHUNK + off, L)] = plsc.bitcast(
                    ipk, jnp.float32)
                st_v[pl.ds(1 * CHUNK + off, L)] = plsc.bitcast(
                    wxp, jnp.float32)
                st_v[pl.ds(2 * CHUNK + off, L)] = plsc.bitcast(
                    wyp, jnp.float32)

            return carry2

        lax.fori_loop(0, CHUNK_ROWS, row_fn, 0)
        pltpu.sync_copy(
            st_v,
            tbl_hbm.at[pl.ds((n * NCHUNKS + ck) * 3 * CHUNK, 3 * CHUNK)])

    def job_fn(q, carry):
        jj = q * NS + sid

        @pl.when(jj < SCJOBS)
        def _():
            table_job(jj)

        return carry

    lax.fori_loop(0, -(-SCJOBS // NS), job_fn, 0)
    plsc.subcore_barrier()

    def t_src(n, ck):
        return tbl_hbm.at[pl.ds((n * NCHUNKS + ck) * 3 * CHUNK, 3 * CHUNK)]

    def compute_chunk(t_v, o0_v, o1_v):
        img1 = img_v.at[pl.ds(HW, HW)]

        @plsc.parallel_loop(0, CVECS, unroll=2)
        def vec_fn(p):
            off = p * L
            ipk = plsc.bitcast(t_v[pl.ds(0 * CHUNK + off, L)], jnp.int32)
            wxp = plsc.bitcast(t_v[pl.ds(1 * CHUNK + off, L)],
                               jnp.bfloat16)
            wyp = plsc.bitcast(t_v[pl.ds(2 * CHUNK + off, L)],
                               jnp.bfloat16)
            wx0, wx1 = plsc.unpack(wxp, format=plsc.PackFormat.INTERLEAVED)
            wy0, wy1 = plsc.unpack(wyp, format=plsc.PackFormat.INTERLEAVED)
            i00 = ipk & 0xFFFF
            dx = lax.shift_right_logical(ipk, 16) & 1
            dyw = lax.shift_right_logical(ipk, 17)
            i01 = i00 + dx
            i10 = i00 + dyw
            i11 = i01 + dyw
            w00 = wx0 * wy0
            w01 = wx1 * wy0
            w10 = wx0 * wy1
            w11 = wx1 * wy1
            a00 = plsc.load_gather(img_v, [i00])
            a01 = plsc.load_gather(img_v, [i01])
            a10 = plsc.load_gather(img_v, [i10])
            a11 = plsc.load_gather(img_v, [i11])
            o0_v[pl.ds(off, L)] = (w00 * a00 + w01 * a01
                                   + w10 * a10 + w11 * a11)
            b00 = plsc.load_gather(img1, [i00])
            b01 = plsc.load_gather(img1, [i01])
            b10 = plsc.load_gather(img1, [i10])
            b11 = plsc.load_gather(img1, [i11])
            o1_v[pl.ds(off, L)] = (w00 * b00 + w01 * b01
                                   + w10 * b10 + w11 * b11)

    def task_fn(t, carry):
        n, chan = task_chan(t)
        pltpu.make_async_copy(img_src(chan), img_v, simg).wait()
        pltpu.async_copy(t_src(n, 0), ta_v, sta)

        def out_dst(ck, ch):
            return out_hbm.at[pl.ds((chan + ch) * HW + ck * CHUNK, CHUNK)]

        def half(k, buf, t_v, t_next, o0_v, o1_v, st_this, st_next,
                 so_this):
            ck = k * 2 + buf
            nxt = jnp.minimum(ck + 1, NCHUNKS - 1)
            pltpu.async_copy(t_src(n, nxt), t_next, st_next)
            pltpu.make_async_copy(t_src(n, ck), t_v, st_this).wait()

            @pl.when(k > 0)
            def _():
                pltpu.make_async_copy(o0_v, out_dst(ck, 0), so_this).wait()
                pltpu.make_async_copy(o1_v, out_dst(ck, 1), so_this).wait()

            compute_chunk(t_v, o0_v, o1_v)
            pltpu.async_copy(o0_v, out_dst(ck, 0), so_this)
            pltpu.async_copy(o1_v, out_dst(ck, 1), so_this)

        def chunk_pair(k, carry2):
            half(k, 0, ta_v, tb_v, oa0_v, oa1_v, sta, stb, soa)
            half(k, 1, tb_v, ta_v, ob0_v, ob1_v, stb, sta, sob)
            return carry2

        lax.fori_loop(0, NCHUNKS // 2, chunk_pair, 0)

        @pl.when(t < TASKS_PER_TILE - 1)
        def _():
            _, chan_next = task_chan(t + 1)
            pltpu.async_copy(img_src(chan_next), img_v, simg)

        pltpu.make_async_copy(t_src(n, NCHUNKS - 1), ta_v, sta).wait()
        pltpu.make_async_copy(oa0_v, out_dst(NCHUNKS - 2, 0), soa).wait()
        pltpu.make_async_copy(oa1_v, out_dst(NCHUNKS - 2, 1), soa).wait()
        pltpu.make_async_copy(ob0_v, out_dst(NCHUNKS - 1, 0), sob).wait()
        pltpu.make_async_copy(ob1_v, out_dst(NCHUNKS - 1, 1), sob).wait()
        return carry

    lax.fori_loop(0, TASKS_PER_TILE, task_fn, 0)


@jax.jit
def _grid_sample_sc(xf, gxy):
    mesh = plsc.VectorSubcoreMesh(core_axis_name="c", subcore_axis_name="s",
                                  num_cores=NC, num_subcores=NS)
    out, _ = pl.kernel(
        _sc_body,
        out_type=(
            jax.ShapeDtypeStruct((N * C * HW,), jnp.float32),
            jax.ShapeDtypeStruct((N * NCHUNKS * 3 * CHUNK,), jnp.float32),
        ),
        mesh=mesh,
        compiler_params=pltpu.CompilerParams(needs_layout_passes=False),
        scratch_types=[
            pltpu.VMEM((2 * HW,), jnp.float32),
            pltpu.VMEM((2 * CHUNK,), jnp.float32),
            pltpu.VMEM((3 * CHUNK,), jnp.float32),
            pltpu.VMEM((3 * CHUNK,), jnp.float32),
            pltpu.VMEM((3 * CHUNK,), jnp.float32),
            pltpu.VMEM((CHUNK,), jnp.float32),
            pltpu.VMEM((CHUNK,), jnp.float32),
            pltpu.VMEM((CHUNK,), jnp.float32),
            pltpu.VMEM((CHUNK,), jnp.float32),
            pltpu.SemaphoreType.DMA,
            pltpu.SemaphoreType.DMA,
            pltpu.SemaphoreType.DMA,
            pltpu.SemaphoreType.DMA,
            pltpu.SemaphoreType.DMA,
        ],
    )(xf, gxy)
    return out


def kernel(x, grid):
    xf = x.reshape(N * C * HW)
    gxy = jnp.stack([grid[..., 0].reshape(N, NCHUNKS, CHUNK),
                     grid[..., 1].reshape(N, NCHUNKS, CHUNK)],
                    axis=2).reshape(N * NCHUNKS * 2 * CHUNK)
    out = _grid_sample_sc(xf, gxy)
    return out.reshape(N, C, H, W)


# speedup vs baseline: 1.3807x; 1.3807x over previous
import jax
import jax.numpy as jnp
from jax import lax
from jax.experimental import pallas as pl
from jax.experimental.pallas import tpu as pltpu
from jax.experimental.pallas import tpu_sc as plsc

N, C, H, W = 4, 96, 224, 224
HW = H * W
L = 16
NC, NS = 2, 16
CPAIRS = C // 2
TASKS_PER_TILE = (2 * CPAIRS) // NS
CHUNK_ROWS = 8
ROW_VECS = W // L
CHUNK = CHUNK_ROWS * W
CVECS = CHUNK // L
NCHUNKS = H // CHUNK_ROWS
SCJOBS = 2 * NCHUNKS
SX = float(W) / float(W - 1)
SY = float(H) / float(H - 1)


def _floor_i32(p):
    t = p.astype(jnp.int32)
    tf = t.astype(jnp.float32)
    return jnp.where(tf > p, t - 1, t)


def _sc_body(x_hbm, gxy_hbm, out_hbm, tbl_hbm, img_v,
             ta_v, tb_v, o0_v, o1_v, simg, sta, stb, so):
    g_v = tb_v.at[pl.ds(0, 2 * CHUNK)]
    st_v = ta_v
    cid = lax.axis_index("c")
    sid = lax.axis_index("s")

    def task_chan(t):
        tt = sid * TASKS_PER_TILE + t
        n_local = tt // CPAIRS
        cp = tt % CPAIRS
        n = cid * 2 + n_local
        return n, n * C + cp * 2

    def img_src(chan):
        return x_hbm.at[pl.ds(chan, 2)]

    _, chan0 = task_chan(0)
    pltpu.async_copy(img_src(chan0), img_v, simg)

    def table_job(jj):
        n_local = jj // NCHUNKS
        ck = jj % NCHUNKS
        n = cid * 2 + n_local
        pltpu.sync_copy(
            gxy_hbm.at[pl.ds((n * NCHUNKS + ck) * 2 * CHUNK, 2 * CHUNK)],
            g_v)

        def row_fn(ri, carry2):
            i = ck * CHUNK_ROWS + ri
            i_f = lax.convert_element_type(i, jnp.float32)

            @plsc.parallel_loop(0, ROW_VECS, unroll=2)
            def vec_fn(v):
                off = ri * W + v * L
                gxv = g_v[pl.ds(off, L)]
                gyv = g_v[pl.ds(CHUNK + off, L)]
                jb = lax.convert_element_type(v * L, jnp.float32)
                jf = lax.iota(jnp.int32, L).astype(jnp.float32) + jb
                px = (gxv + jf) * SX - 0.5
                py = (gyv + i_f) * SY - 0.5
                px = jnp.minimum(jnp.maximum(px, -2.0), float(W + 2))
                py = jnp.minimum(jnp.maximum(py, -2.0), float(H + 2))
                x0 = _floor_i32(px)
                y0 = _floor_i32(py)
                wx1 = px - x0.astype(jnp.float32)
                wx0 = 1.0 - wx1
                wy1 = py - y0.astype(jnp.float32)
                wy0 = 1.0 - wy1
                zero = jnp.zeros((L,), jnp.float32)
                wx0 = jnp.where((x0 >= 0) & (x0 <= W - 1), wx0, zero)
                wx1 = jnp.where((x0 >= -1) & (x0 <= W - 2), wx1, zero)
                wy0 = jnp.where((y0 >= 0) & (y0 <= H - 1), wy0, zero)
                wy1 = jnp.where((y0 >= -1) & (y0 <= H - 2), wy1, zero)
                xc0 = jnp.maximum(jnp.minimum(x0, W - 1), 0)
                xc1 = jnp.maximum(jnp.minimum(x0 + 1, W - 1), 0)
                yc0 = jnp.maximum(jnp.minimum(y0, H - 1), 0)
                yc1 = jnp.maximum(jnp.minimum(y0 + 1, H - 1), 0)
                ipk = (xc0 + lax.shift_left(yc0, 8)
                       + lax.shift_left(xc1 - xc0, 16)
                       + lax.shift_left(yc1 - yc0, 17))
                wxp = plsc.pack(wx0, wx1, format=plsc.PackFormat.INTERLEAVED)
                wyp = plsc.pack(wy0, wy1, format=plsc.PackFormat.INTERLEAVED)
                st_v[pl.ds(0 * CHUNK + off, L)] = plsc.bitcast(
                    ipk, jnp.float32)
                st_v[pl.ds(1 * CHUNK + off, L)] = plsc.bitcast(
                    wxp, jnp.float32)
                st_v[pl.ds(2 * CHUNK + off, L)] = plsc.bitcast(
                    wyp, jnp.float32)

            return carry2

        lax.fori_loop(0, CHUNK_ROWS, row_fn, 0)
        pltpu.sync_copy(
            st_v,
            tbl_hbm.at[pl.ds((n * NCHUNKS + ck) * 3 * CHUNK, 3 * CHUNK)])

    def job_fn(q, carry):
        jj = q * NS + sid

        @pl.when(jj < SCJOBS)
        def _():
            table_job(jj)

        return carry

    lax.fori_loop(0, -(-SCJOBS // NS), job_fn, 0)
    plsc.subcore_barrier()

    def t_src(n, ck):
        return tbl_hbm.at[pl.ds((n * NCHUNKS + ck) * 3 * CHUNK, 3 * CHUNK)]

    def compute_chunk(t_v, o0_v, o1_v):
        img0 = img_v.at[0]
        img1 = img_v.at[1]

        @plsc.parallel_loop(0, CVECS, unroll=2)
        def vec_fn(p):
            ri = p // ROW_VECS
            jv = (p % ROW_VECS) * L
            off = p * L
            ipk = plsc.bitcast(t_v[pl.ds(0 * CHUNK + off, L)], jnp.int32)
            wxp = plsc.bitcast(t_v[pl.ds(1 * CHUNK + off, L)],
                               jnp.bfloat16)
            wyp = plsc.bitcast(t_v[pl.ds(2 * CHUNK + off, L)],
                               jnp.bfloat16)
            wx0, wx1 = plsc.unpack(wxp, format=plsc.PackFormat.INTERLEAVED)
            wy0, wy1 = plsc.unpack(wyp, format=plsc.PackFormat.INTERLEAVED)
            x0 = ipk & 0xFF
            y0 = lax.shift_right_logical(ipk, 8) & 0xFF
            dx = lax.shift_right_logical(ipk, 16) & 1
            dy = lax.shift_right_logical(ipk, 17)
            x1 = x0 + dx
            y1 = y0 + dy
            w00 = wx0 * wy0
            w01 = wx1 * wy0
            w10 = wx0 * wy1
            w11 = wx1 * wy1
            a00 = plsc.load_gather(img0, [y0, x0])
            a01 = plsc.load_gather(img0, [y0, x1])
            a10 = plsc.load_gather(img0, [y1, x0])
            a11 = plsc.load_gather(img0, [y1, x1])
            o0_v[0, ri, pl.ds(jv, L)] = (w00 * a00 + w01 * a01
                                         + w10 * a10 + w11 * a11)
            b00 = plsc.load_gather(img1, [y0, x0])
            b01 = plsc.load_gather(img1, [y0, x1])
            b10 = plsc.load_gather(img1, [y1, x0])
            b11 = plsc.load_gather(img1, [y1, x1])
            o1_v[0, ri, pl.ds(jv, L)] = (w00 * b00 + w01 * b01
                                         + w10 * b10 + w11 * b11)

    def task_fn(t, carry):
        n, chan = task_chan(t)
        pltpu.make_async_copy(img_src(chan), img_v, simg).wait()
        pltpu.async_copy(t_src(n, 0), ta_v, sta)

        def out_dst(ck, ch):
            return out_hbm.at[pl.ds(chan + ch, 1),
                              pl.ds(ck * CHUNK_ROWS, CHUNK_ROWS)]

        def half(k, buf, t_v, t_next, st_this, st_next, wait_out):
            ck = k * 2 + buf
            nxt = jnp.minimum(ck + 1, NCHUNKS - 1)
            pltpu.async_copy(t_src(n, nxt), t_next, st_next)
            pltpu.make_async_copy(t_src(n, ck), t_v, st_this).wait()

            @pl.when(wait_out)
            def _():
                pltpu.make_async_copy(o0_v, out_dst(ck, 0), so).wait()
                pltpu.make_async_copy(o1_v, out_dst(ck, 1), so).wait()

            compute_chunk(t_v, o0_v, o1_v)
            pltpu.async_copy(o0_v, out_dst(ck, 0), so)
            pltpu.async_copy(o1_v, out_dst(ck, 1), so)

        def chunk_pair(k, carry2):
            half(k, 0, ta_v, tb_v, sta, stb, k > 0)
            half(k, 1, tb_v, ta_v, stb, sta, True)
            return carry2

        lax.fori_loop(0, NCHUNKS // 2, chunk_pair, 0)

        @pl.when(t < TASKS_PER_TILE - 1)
        def _():
            _, chan_next = task_chan(t + 1)
            pltpu.async_copy(img_src(chan_next), img_v, simg)

        pltpu.make_async_copy(t_src(n, NCHUNKS - 1), ta_v, sta).wait()
        pltpu.make_async_copy(o0_v, out_dst(NCHUNKS - 1, 0), so).wait()
        pltpu.make_async_copy(o1_v, out_dst(NCHUNKS - 1, 1), so).wait()
        return carry

    lax.fori_loop(0, TASKS_PER_TILE, task_fn, 0)


@jax.jit
def _grid_sample_sc(xf, gxy):
    mesh = plsc.VectorSubcoreMesh(core_axis_name="c", subcore_axis_name="s",
                                  num_cores=NC, num_subcores=NS)
    out, _ = pl.kernel(
        _sc_body,
        out_type=(
            jax.ShapeDtypeStruct((N * C, H, W), jnp.float32),
            jax.ShapeDtypeStruct((N * NCHUNKS * 3 * CHUNK,), jnp.float32),
        ),
        mesh=mesh,
        compiler_params=pltpu.CompilerParams(needs_layout_passes=False),
        scratch_types=[
            pltpu.VMEM((2, H, W), jnp.float32),
            pltpu.VMEM((3 * CHUNK,), jnp.float32),
            pltpu.VMEM((3 * CHUNK,), jnp.float32),
            pltpu.VMEM((1, CHUNK_ROWS, W), jnp.float32),
            pltpu.VMEM((1, CHUNK_ROWS, W), jnp.float32),
            pltpu.SemaphoreType.DMA,
            pltpu.SemaphoreType.DMA,
            pltpu.SemaphoreType.DMA,
            pltpu.SemaphoreType.DMA,
        ],
    )(xf, gxy)
    return out


def kernel(x, grid):
    xf = x.reshape(N * C, H, W)
    gxy = jnp.stack([grid[..., 0].reshape(N, NCHUNKS, CHUNK),
                     grid[..., 1].reshape(N, NCHUNKS, CHUNK)],
                    axis=2).reshape(N * NCHUNKS * 2 * CHUNK)
    out = _grid_sample_sc(xf, gxy)
    return out.reshape(N, C, H, W)
